# baseline (speedup 1.0000x reference)
import functools

import jax
import jax.numpy as jnp
from jax import lax
from jax.experimental import pallas as pl
from jax.experimental.pallas import tpu as pltpu
from jax.experimental.pallas import tpu_sc as plsc

_BASE_HU = -2.0
_BASE_NORM = 0.0

_NW = 32
_CH = 8192


def _cumsum16(x, tmp_v, iot):
    acc = x
    for s in (1, 2, 4, 8):
        tmp_v[...] = acc
        t = plsc.load_gather(tmp_v, [jnp.maximum(iot - s, 0)])
        acc = acc + jnp.where(iot >= s, t, 0.0)
    return acc


def _build_tables(hu_v, norm_v, bt_v, a_v, c_v, tmp_v):
    iot = lax.iota(jnp.int32, 16)
    inf = jnp.float32(jnp.inf)
    for j in range(4):
        habs = jnp.abs(hu_v[j])
        nabs = jnp.abs(norm_v[j])
        H = _cumsum16(habs, tmp_v, iot)
        N = _cumsum16(nabs, tmp_v, iot)
        Hprev = H - habs
        Nprev = N - nabs
        k = nabs / habs
        mid = (iot >= 1) & (iot <= 7)
        avec = jnp.where(mid, k, 0.0)
        cin = Nprev - k * Hprev
        cvec = jnp.where(iot == 8, Nprev + _BASE_NORM, jnp.where(mid, cin, 0.0))
        bvec = jnp.where(iot <= 7, _BASE_HU + H, inf)
        bt_v[j] = bvec
        a_v[j] = avec
        c_v[j] = cvec


def _make_body(nch):
    def _sc_body(img_hbm, hu_hbm, norm_hbm, out_hbm, hu_v, norm_v, bt_v, a_v, c_v, tmp_v, x_v, y_v):
        c = lax.axis_index("c")
        s = lax.axis_index("s")
        wid = s * 2 + c
        b = wid // 16
        col0 = (wid % 16) * (_CH * nch)

        pltpu.sync_copy(hu_hbm, hu_v)
        pltpu.sync_copy(norm_hbm, norm_v)
        _build_tables(hu_v, norm_v, bt_v, a_v, c_v, tmp_v)

        jsplat = [jnp.full((16,), j, jnp.int32) for j in range(4)]

        def chunk_body(g, _):
            col = col0 + g * _CH
            pltpu.sync_copy(img_hbm.at[b, pl.ds(col, _CH)], x_v)

            def vec_body(i, _):
                off = i * 16
                x = x_v[pl.ds(off, 16)]
                for j in range(4):
                    p = jnp.zeros((16,), jnp.int32)
                    for step in (8, 4, 2, 1):
                        bb = plsc.load_gather(bt_v, [jsplat[j], p + (step - 1)])
                        p = jnp.where(x >= bb, p + step, p)
                    a = plsc.load_gather(a_v, [jsplat[j], p])
                    cc = plsc.load_gather(c_v, [jsplat[j], p])
                    y_v[j, pl.ds(off, 16)] = a * x + cc
                return 0

            lax.fori_loop(0, _CH // 16, vec_body, 0)
            for j in range(4):
                pltpu.sync_copy(y_v.at[j], out_hbm.at[b * 4 + j, pl.ds(col, _CH)])
            return 0

        lax.fori_loop(0, nch, chunk_body, 0)

    return _sc_body


def kernel(img, hu_lis, norm_lis):
    B, C, D, H, W = img.shape
    M = D * H * W
    total = B * M
    pw = total // _NW
    nch = pw // _CH
    assert pw % _CH == 0 and M % pw == 0

    x = img.reshape(B, M)
    hu16 = jnp.pad(hu_lis, ((0, 0), (0, 8)))
    norm16 = jnp.pad(norm_lis, ((0, 0), (0, 8)))

    mesh = plsc.VectorSubcoreMesh(core_axis_name="c", subcore_axis_name="s")
    f = pl.kernel(
        _make_body(nch),
        out_type=jax.ShapeDtypeStruct((B * 4, M), jnp.float32),
        mesh=mesh,
        scratch_types=[
            pltpu.VMEM((4, 16), jnp.float32),
            pltpu.VMEM((4, 16), jnp.float32),
            pltpu.VMEM((4, 16), jnp.float32),
            pltpu.VMEM((4, 16), jnp.float32),
            pltpu.VMEM((4, 16), jnp.float32),
            pltpu.VMEM((16,), jnp.float32),
            pltpu.VMEM((_CH,), jnp.float32),
            pltpu.VMEM((4, _CH), jnp.float32),
        ],
        compiler_params=pltpu.CompilerParams(needs_layout_passes=False),
    )
    out = f(x, hu16, norm16)
    return out.reshape(B, 4, D, H, W)


# speedup vs baseline: 2.2025x; 2.2025x over previous
import functools

import jax
import jax.numpy as jnp
from jax import lax
from jax.experimental import pallas as pl
from jax.experimental.pallas import tpu as pltpu
from jax.experimental.pallas import tpu_sc as plsc

_BASE_HU = -2.0
_BASE_NORM = 0.0

_NW = 32
_CH = 8192
_UNROLL = 4


def _cumsum16(x, tmp_v, iot):
    acc = x
    for s in (1, 2, 4, 8):
        tmp_v[...] = acc
        t = plsc.load_gather(tmp_v, [jnp.maximum(iot - s, 0)])
        acc = acc + jnp.where(iot >= s, t, 0.0)
    return acc


def _build_tables(hu_v, norm_v, bt_v, a_v, c_v, tmp_v):
    iot = lax.iota(jnp.int32, 16)
    inf = jnp.float32(jnp.inf)
    for j in range(4):
        habs = jnp.abs(hu_v[j])
        nabs = jnp.abs(norm_v[j])
        H = _cumsum16(habs, tmp_v, iot)
        N = _cumsum16(nabs, tmp_v, iot)
        Hprev = H - habs
        Nprev = N - nabs
        k = nabs / habs
        mid = (iot >= 1) & (iot <= 7)
        avec = jnp.where(mid, k, 0.0)
        cin = Nprev - k * Hprev
        cvec = jnp.where(iot == 8, Nprev + _BASE_NORM, jnp.where(mid, cin, 0.0))
        bvec = jnp.where(iot <= 7, _BASE_HU + H, inf)
        bt_v[j] = bvec
        a_v[j] = avec
        c_v[j] = cvec


def _splats(bt_v, a_v, c_v, j):
    jv = jnp.full((16,), j, jnp.int32)
    bs = [plsc.load_gather(bt_v, [jv, jnp.full((16,), i, jnp.int32)]) for i in range(8)]
    as_ = [plsc.load_gather(a_v, [jv, jnp.full((16,), i, jnp.int32)]) for i in range(1, 8)]
    cs = [plsc.load_gather(c_v, [jv, jnp.full((16,), i, jnp.int32)]) for i in range(1, 8)]
    top = plsc.load_gather(c_v, [jv, jnp.full((16,), 8, jnp.int32)])
    return bs, as_, cs, top


def _make_body(nch):
    def _sc_body(img_hbm, hu_hbm, norm_hbm, out_hbm,
                 hu_v, norm_v, bt_v, a_v, c_v, tmp_v, x_v, y_v,
                 in_sems, out_sems):
        c = lax.axis_index("c")
        s = lax.axis_index("s")
        wid = s * 2 + c
        b = wid // 16
        col0 = (wid % 16) * (_CH * nch)

        pltpu.sync_copy(hu_hbm, hu_v)
        pltpu.sync_copy(norm_hbm, norm_v)
        _build_tables(hu_v, norm_v, bt_v, a_v, c_v, tmp_v)

        def in_copy(g, buf):
            return pltpu.make_async_copy(
                img_hbm.at[b, pl.ds(col0 + g * _CH, _CH)], x_v.at[buf], in_sems.at[buf])

        def out_copy(g, buf, j):
            return pltpu.make_async_copy(
                y_v.at[buf, j], out_hbm.at[b * 4 + j, pl.ds(col0 + g * _CH, _CH)],
                out_sems.at[buf])

        in_copy(0, 0).start()

        def chunk_body(g, _):
            buf = lax.rem(g, 2)
            in_copy(g, buf).wait()

            @pl.when(g + 1 < nch)
            def _():
                in_copy(g + 1, lax.rem(g + 1, 2)).start()

            @pl.when(g >= 2)
            def _():
                for j in range(4):
                    out_copy(g - 2, buf, j).wait()

            for j in range(4):
                bs, as_, cs, top = _splats(bt_v, a_v, c_v, j)

                def vec_body(i, _, buf=buf, j=j, bs=bs, as_=as_, cs=cs, top=top):
                    for u in range(_UNROLL):
                        off = (i * _UNROLL + u) * 16
                        x = x_v[buf, pl.ds(off, 16)]
                        y = jnp.zeros((16,), jnp.float32)
                        for t in range(7):
                            y = jnp.where(x >= bs[t], as_[t] * x + cs[t], y)
                        y = jnp.where(x >= bs[7], top, y)
                        y_v[buf, j, pl.ds(off, 16)] = y
                    return 0

                lax.fori_loop(0, _CH // (16 * _UNROLL), vec_body, 0)
                out_copy(g, buf, j).start()
            return 0

        lax.fori_loop(0, nch, chunk_body, 0)
        for g, buf in ((nch - 2, 0), (nch - 1, 1)):
            for j in range(4):
                out_copy(g, buf, j).wait()

    return _sc_body


def kernel(img, hu_lis, norm_lis):
    B, C, D, H, W = img.shape
    M = D * H * W
    total = B * M
    pw = total // _NW
    nch = pw // _CH
    assert pw % _CH == 0 and M % pw == 0

    x = img.reshape(B, M)
    hu16 = jnp.pad(hu_lis, ((0, 0), (0, 8)))
    norm16 = jnp.pad(norm_lis, ((0, 0), (0, 8)))

    mesh = plsc.VectorSubcoreMesh(core_axis_name="c", subcore_axis_name="s")
    f = pl.kernel(
        _make_body(nch),
        out_type=jax.ShapeDtypeStruct((B * 4, M), jnp.float32),
        mesh=mesh,
        scratch_types=[
            pltpu.VMEM((4, 16), jnp.float32),
            pltpu.VMEM((4, 16), jnp.float32),
            pltpu.VMEM((4, 16), jnp.float32),
            pltpu.VMEM((4, 16), jnp.float32),
            pltpu.VMEM((4, 16), jnp.float32),
            pltpu.VMEM((16,), jnp.float32),
            pltpu.VMEM((2, _CH), jnp.float32),
            pltpu.VMEM((2, 4, _CH), jnp.float32),
            pltpu.SemaphoreType.DMA((2,)),
            pltpu.SemaphoreType.DMA((2,)),
        ],
        compiler_params=pltpu.CompilerParams(needs_layout_passes=False),
    )
    out = f(x, hu16, norm16)
    return out.reshape(B, 4, D, H, W)
